# baseline (speedup 1.0000x reference)
import jax
import jax.numpy as jnp
from jax import lax
from jax.experimental import pallas as pl
from jax.experimental.pallas import tpu as pltpu

_ORDER = 50
_EMBED = 64
_LANES = 128
_STEPS = 5
_K = _ORDER // _STEPS


def _block_index_map(k):
    def index_map(i, idx_ref, pos_ref):
        return 0, jnp.maximum(idx_ref[i * _K + k], 0) // _LANES
    return index_map


def _body(idx_ref, pos_ref, *rest):
    blocks = rest[:_K]
    out_v = rest[_K]
    acc_v = rest[_K + 1]
    i = pl.program_id(0)

    lane = lax.broadcasted_iota(jnp.int32, (1, _LANES), 1)
    acc = jnp.zeros((_EMBED, _LANES), jnp.float32)
    for k in range(_K):
        row = idx_ref[i * _K + k]
        wi = jnp.where(row != -1, pos_ref[i * _K + k], jnp.float32(0.0))
        rem = lax.rem(jnp.maximum(row, 0), _LANES)
        sel = jnp.where(lane == rem, wi, jnp.float32(0.0))
        acc = acc + blocks[k][...] * sel

    @pl.when(i == 0)
    def _():
        acc_v[...] = acc

    @pl.when(i > 0)
    def _():
        acc_v[...] = acc_v[...] + acc

    @pl.when(i == _STEPS - 1)
    def _():
        denom = jnp.float32(0.0)
        for k in range(_ORDER):
            denom = denom + jnp.where(idx_ref[k] != -1, pos_ref[k],
                                      jnp.float32(0.0))
        out_v[...] = jnp.sum(acc_v[...], axis=1, keepdims=True) / denom


@jax.jit
def kernel(partial_path_candidate, objects_embeds, positional_encoding):
    table_t = objects_embeds.T
    grid_spec = pltpu.PrefetchScalarGridSpec(
        num_scalar_prefetch=2,
        grid=(_STEPS,),
        in_specs=[
            pl.BlockSpec((_EMBED, _LANES), _block_index_map(k))
            for k in range(_K)
        ],
        out_specs=pl.BlockSpec((_EMBED, 1), lambda i, idx_ref, pos_ref: (0, 0)),
        scratch_shapes=[pltpu.VMEM((_EMBED, _LANES), jnp.float32)],
    )
    out = pl.pallas_call(
        _body,
        grid_spec=grid_spec,
        out_shape=jax.ShapeDtypeStruct((_EMBED, 1), jnp.float32),
    )(partial_path_candidate, positional_encoding, *([table_t] * _K))
    return out.reshape(_EMBED)


# speedup vs baseline: 1.4264x; 1.4264x over previous
import jax
import jax.numpy as jnp
from jax import lax
from jax.experimental import pallas as pl
from jax.experimental.pallas import tpu as pltpu

_ORDER = 50
_EMBED = 64
_LANES = 128


def _body(idx_s, pos_s, table_t, out_v, rows_v, sem):
    copies = []
    for k in range(_ORDER):
        grp = jax.lax.shift_right_logical(jnp.maximum(idx_s[k], 0), 7)
        copies.append(pltpu.make_async_copy(
            table_t.at[:, pl.ds(grp * _LANES, _LANES)],
            rows_v.at[pl.ds(k * _EMBED, _EMBED), :], sem))
    for cp in copies:
        cp.start()

    lane = lax.broadcasted_iota(jnp.int32, (1, _LANES), 1)

    for cp in copies:
        cp.wait()

    acc = jnp.zeros((_EMBED, _LANES), jnp.float32)
    denom = jnp.float32(0.0)
    for k in range(_ORDER):
        row = idx_s[k]
        wi = jnp.where(row != -1, pos_s[k], jnp.float32(0.0))
        denom = denom + wi
        rem = jnp.maximum(row, 0) & (_LANES - 1)
        sel = jnp.where(lane == rem, wi, jnp.float32(0.0))
        acc = acc + rows_v[pl.ds(k * _EMBED, _EMBED), :] * sel

    out_v[...] = jnp.sum(acc, axis=1, keepdims=True) / denom


@jax.jit
def kernel(partial_path_candidate, objects_embeds, positional_encoding):
    table_t = objects_embeds.T
    out = pl.pallas_call(
        _body,
        out_shape=jax.ShapeDtypeStruct((_EMBED, 1), jnp.float32),
        in_specs=[
            pl.BlockSpec(memory_space=pltpu.SMEM),
            pl.BlockSpec(memory_space=pltpu.SMEM),
            pl.BlockSpec(memory_space=pl.ANY),
        ],
        out_specs=pl.BlockSpec(memory_space=pltpu.VMEM),
        scratch_shapes=[
            pltpu.VMEM((_ORDER * _EMBED, _LANES), jnp.float32),
            pltpu.SemaphoreType.DMA,
        ],
    )(partial_path_candidate, positional_encoding, table_t)
    return out.reshape(_EMBED)


# speedup vs baseline: 1.4887x; 1.0437x over previous
import jax
import jax.numpy as jnp
from jax import lax
from jax.experimental import pallas as pl
from jax.experimental.pallas import tpu as pltpu

_ORDER = 50
_EMBED = 64
_LANES = 128


def _body(idx_s, pos_s, table_t, out_v, rows_v, sem):
    copies = []
    for k in range(_ORDER):
        grp = jax.lax.shift_right_logical(jnp.maximum(idx_s[k], 0), 7)
        copies.append(pltpu.make_async_copy(
            table_t.at[:, pl.ds(grp * _LANES, _LANES)],
            rows_v.at[pl.ds(k * _EMBED, _EMBED), :], sem))
    for cp in copies:
        cp.start()

    lane = lax.broadcasted_iota(jnp.int32, (1, _LANES), 1)

    wis, rems = [], []
    denom = jnp.float32(0.0)
    for k in range(_ORDER):
        row = idx_s[k]
        wi = jnp.where(row != -1, pos_s[k], jnp.float32(0.0))
        denom = denom + wi
        wis.append(wi)
        rems.append(jnp.maximum(row, 0) & (_LANES - 1))

    acc = jnp.zeros((_EMBED, _LANES), jnp.float32)
    for k in range(_ORDER):
        copies[k].wait()
        sel = jnp.where(lane == rems[k], wis[k], jnp.float32(0.0))
        acc = acc + rows_v[pl.ds(k * _EMBED, _EMBED), :] * sel

    out_v[...] = jnp.sum(acc, axis=1, keepdims=True) / denom


@jax.jit
def kernel(partial_path_candidate, objects_embeds, positional_encoding):
    table_t = objects_embeds.T
    out = pl.pallas_call(
        _body,
        out_shape=jax.ShapeDtypeStruct((_EMBED, 1), jnp.float32),
        in_specs=[
            pl.BlockSpec(memory_space=pltpu.SMEM),
            pl.BlockSpec(memory_space=pltpu.SMEM),
            pl.BlockSpec(memory_space=pl.ANY),
        ],
        out_specs=pl.BlockSpec(memory_space=pltpu.VMEM),
        scratch_shapes=[
            pltpu.VMEM((_ORDER * _EMBED, _LANES), jnp.float32),
            pltpu.SemaphoreType.DMA,
        ],
    )(partial_path_candidate, positional_encoding, table_t)
    return out.reshape(_EMBED)
